# Initial kernel scaffold; baseline (speedup 1.0000x reference)
import jax
import jax.numpy as jnp
from jax.experimental import pallas as pl


def kernel(x, edge_index, batch, graph_attr, W1, b1, W2, b2, W3, b3, g1, be1, g2, be2, g3, be3, fc1_W, fc1_b, fc2_W, fc2_b):
    raise NotImplementedError("write your pallas kernel here")


# speedup vs baseline: 10.8207x; 10.8207x over previous
import functools

import jax
import jax.numpy as jnp
from jax import lax
from jax.experimental import pallas as pl
from jax.experimental.pallas import tpu as pltpu
from jax.experimental.pallas import tpu_sc as plsc

_NC = 2
_NS = 16
_LN_EPS = 1e-5
_WROWS = 624


def _zero_slice(acc_sh, zbuf, r0, nrows):
    done = 0
    while done + 128 <= nrows:
        pltpu.sync_copy(zbuf, acc_sh.at[pl.ds(r0 + done, 128)])
        done += 128
    if done < nrows:
        pltpu.sync_copy(zbuf.at[pl.ds(0, nrows - done)],
                        acc_sh.at[pl.ds(r0 + done, nrows - done)])


def _make_deg_kernel(N, E):
    NW = _NC * _NS
    EPW = E // NW
    C = 80
    NCHUNK = EPW // C
    TAIL0 = _NS * _WROWS
    TAIL = N - TAIL0
    mesh = plsc.VectorSubcoreMesh(core_axis_name="c", subcore_axis_name="s")

    @functools.partial(
        pl.kernel,
        out_type=jax.ShapeDtypeStruct((_NC, N, 16), jnp.float32),
        mesh=mesh,
        scratch_types=[
            pltpu.VMEM_SHARED((N, 16), jnp.float32),
            pltpu.VMEM((C,), jnp.int32),
            pltpu.VMEM((C, 16), jnp.float32),
            pltpu.VMEM((128, 16), jnp.float32),
        ],
    )
    def deg_kernel(dst_hbm, out_hbm, acc_sh, didx, ones_v, zbuf):
        c = lax.axis_index("c")
        s = lax.axis_index("s")
        w = c * _NS + s

        def fill(i, _):
            zbuf[i, :] = jnp.zeros((16,), jnp.float32)
            ones_v[lax.rem(i, C), :] = jnp.ones((16,), jnp.float32)
            return 0

        lax.fori_loop(0, 128, fill, 0)

        r0 = s * _WROWS
        _zero_slice(acc_sh, zbuf, r0, _WROWS)

        @pl.when(s == _NS - 1)
        def _ztail():
            _zero_slice(acc_sh, zbuf, TAIL0, TAIL)

        plsc.subcore_barrier()

        e0 = w * EPW

        def body(i, _):
            base = pl.multiple_of(e0 + i * C, 8)
            pltpu.sync_copy(dst_hbm.at[pl.ds(base, C)], didx)
            pltpu.sync_copy(ones_v, acc_sh.at[didx], add=True)
            return 0

        lax.fori_loop(0, NCHUNK, body, 0)
        plsc.subcore_barrier()
        pltpu.sync_copy(acc_sh.at[pl.ds(r0, _WROWS)],
                        out_hbm.at[c, pl.ds(r0, _WROWS)])

        @pl.when(s == _NS - 1)
        def _wtail():
            pltpu.sync_copy(acc_sh.at[pl.ds(TAIL0, TAIL)],
                            out_hbm.at[c, pl.ds(TAIL0, TAIL)])

    return deg_kernel


def _make_agg_kernel(N, E, D):
    NW = _NC * _NS
    EPW = E // NW
    C = 80
    NCHUNK = EPW // C
    TAIL0 = _NS * _WROWS
    TAIL = N - TAIL0
    mesh = plsc.VectorSubcoreMesh(core_axis_name="c", subcore_axis_name="s")

    @functools.partial(
        pl.kernel,
        out_type=jax.ShapeDtypeStruct((_NC, N, D), jnp.float32),
        mesh=mesh,
        scratch_types=[
            pltpu.VMEM_SHARED((N, D), jnp.float32),
            pltpu.VMEM((C,), jnp.int32),
            pltpu.VMEM((C,), jnp.int32),
            pltpu.VMEM((C, D), jnp.float32),
            pltpu.VMEM((128, D), jnp.float32),
            pltpu.SemaphoreType.DMA,
        ],
    )
    def agg_kernel(u_hbm, src_hbm, dst_hbm, out_hbm,
                   acc_sh, sidx, didx, rows, zbuf, sem):
        c = lax.axis_index("c")
        s = lax.axis_index("s")
        w = c * _NS + s

        def fill(i, _):
            for k in range(D // 16):
                zbuf[i, pl.ds(16 * k, 16)] = jnp.zeros((16,), jnp.float32)
            return 0

        lax.fori_loop(0, 128, fill, 0)

        r0 = s * _WROWS
        _zero_slice(acc_sh, zbuf, r0, _WROWS)

        @pl.when(s == _NS - 1)
        def _ztail():
            _zero_slice(acc_sh, zbuf, TAIL0, TAIL)

        plsc.subcore_barrier()

        e0 = w * EPW

        def body(i, _):
            base = pl.multiple_of(e0 + i * C, 8)
            pltpu.sync_copy(src_hbm.at[pl.ds(base, C)], sidx)
            pltpu.sync_copy(dst_hbm.at[pl.ds(base, C)], didx)
            pltpu.async_copy(u_hbm.at[sidx], rows, sem).wait()
            pltpu.sync_copy(rows, acc_sh.at[didx], add=True)
            return 0

        lax.fori_loop(0, NCHUNK, body, 0)
        plsc.subcore_barrier()
        pltpu.sync_copy(acc_sh.at[pl.ds(r0, _WROWS)],
                        out_hbm.at[c, pl.ds(r0, _WROWS)])

        @pl.when(s == _NS - 1)
        def _wtail():
            pltpu.sync_copy(acc_sh.at[pl.ds(TAIL0, TAIL)],
                            out_hbm.at[c, pl.ds(TAIL0, TAIL)])

    return agg_kernel


def _dinv_from_parts(degp):
    deg = degp[0, :, 0] + degp[1, :, 0] + 1.0
    return lax.rsqrt(deg)[:, None]


def _pre_tc(x, W, degp, R=1000):
    N, D = x.shape

    def body(x_ref, w_ref, degp_ref, o_ref):
        dinv = _dinv_from_parts(degp_ref[...])
        o_ref[...] = jnp.dot(x_ref[...], w_ref[...],
                             preferred_element_type=jnp.float32) * dinv

    return pl.pallas_call(
        body,
        grid=(N // R,),
        in_specs=[
            pl.BlockSpec((R, D), lambda i: (i, 0)),
            pl.BlockSpec((D, D), lambda i: (0, 0)),
            pl.BlockSpec((2, R, 16), lambda i: (0, i, 0)),
        ],
        out_specs=pl.BlockSpec((R, D), lambda i: (i, 0)),
        out_shape=jax.ShapeDtypeStruct((N, D), jnp.float32),
    )(x, W, degp)


def _layer_post(t, g_v, be_v):
    mu = jnp.mean(t, axis=-1, keepdims=True)
    tcen = t - mu
    var = jnp.mean(tcen * tcen, axis=-1, keepdims=True)
    y = tcen * lax.rsqrt(var + _LN_EPS) * g_v + be_v
    return jnp.maximum(y, 0.0)


def _mid_tc(aggp, u, degp, b, g, be, Wn, R=1000):
    N, D = u.shape

    def body(aggp_ref, u_ref, degp_ref, b_ref, g_ref, be_ref, w_ref, o_ref):
        dinv = _dinv_from_parts(degp_ref[...])
        a = aggp_ref[...]
        t = dinv * (a[0] + a[1] + u_ref[...]) + b_ref[...]
        h = _layer_post(t, g_ref[...], be_ref[...])
        o_ref[...] = jnp.dot(h, w_ref[...],
                             preferred_element_type=jnp.float32) * dinv

    return pl.pallas_call(
        body,
        grid=(N // R,),
        in_specs=[
            pl.BlockSpec((2, R, D), lambda i: (0, i, 0)),
            pl.BlockSpec((R, D), lambda i: (i, 0)),
            pl.BlockSpec((2, R, 16), lambda i: (0, i, 0)),
            pl.BlockSpec((D,), lambda i: (0,)),
            pl.BlockSpec((D,), lambda i: (0,)),
            pl.BlockSpec((D,), lambda i: (0,)),
            pl.BlockSpec((D, D), lambda i: (0, 0)),
        ],
        out_specs=pl.BlockSpec((R, D), lambda i: (i, 0)),
        out_shape=jax.ShapeDtypeStruct((N, D), jnp.float32),
    )(aggp, u, degp, b, g, be, Wn)


def _final_tc(aggp, u, degp, b, g, be, batch, graph_attr,
              fc1_W, fc1_b, fc2_W, fc2_b, R=1000):
    N, D = u.shape
    B, G = graph_attr.shape
    OUT = fc2_W.shape[1]
    nblk = N // R

    def body(aggp_ref, u_ref, degp_ref, b_ref, g_ref, be_ref, batch_ref,
             ga_ref, fc1w_ref, fc1b_ref, fc2w_ref, fc2b_ref, o_ref,
             sum_ref, cnt_ref):
        i = pl.program_id(0)

        @pl.when(i == 0)
        def _init():
            sum_ref[...] = jnp.zeros_like(sum_ref)
            cnt_ref[...] = jnp.zeros_like(cnt_ref)

        dinv = _dinv_from_parts(degp_ref[...])
        a = aggp_ref[...]
        t = dinv * (a[0] + a[1] + u_ref[...]) + b_ref[...]
        h = _layer_post(t, g_ref[...], be_ref[...])

        batch_blk = batch_ref[...].reshape(R)
        onehot = (batch_blk[:, None] ==
                  lax.broadcasted_iota(jnp.int32, (R, B), 1)
                  ).astype(jnp.float32)
        sum_ref[...] += lax.dot_general(onehot, h, (((0,), (0,)), ((), ())),
                                        preferred_element_type=jnp.float32)
        cnt_ref[...] += jnp.sum(onehot, axis=0)[None, :]

        @pl.when(i == nblk - 1)
        def _fin():
            pooled = sum_ref[...] / jnp.maximum(cnt_ref[0, :], 1.0)[:, None]
            fc1w = fc1w_ref[...]
            z = (jnp.dot(pooled, fc1w[:D], preferred_element_type=jnp.float32)
                 + jnp.dot(ga_ref[...], fc1w[D:],
                           preferred_element_type=jnp.float32)
                 + fc1b_ref[...])
            z = jnp.maximum(z, 0.0)
            o_ref[...] = (jnp.dot(z, fc2w_ref[...],
                                  preferred_element_type=jnp.float32)
                          + fc2b_ref[...])

    return pl.pallas_call(
        body,
        grid=(nblk,),
        in_specs=[
            pl.BlockSpec((2, R, D), lambda i: (0, i, 0)),
            pl.BlockSpec((R, D), lambda i: (i, 0)),
            pl.BlockSpec((2, R, 16), lambda i: (0, i, 0)),
            pl.BlockSpec((D,), lambda i: (0,)),
            pl.BlockSpec((D,), lambda i: (0,)),
            pl.BlockSpec((D,), lambda i: (0,)),
            pl.BlockSpec((1, 1, R), lambda i: (i, 0, 0)),
            pl.BlockSpec((B, G), lambda i: (0, 0)),
            pl.BlockSpec(fc1_W.shape, lambda i: (0, 0)),
            pl.BlockSpec((D,), lambda i: (0,)),
            pl.BlockSpec((D, OUT), lambda i: (0, 0)),
            pl.BlockSpec((OUT,), lambda i: (0,)),
        ],
        out_specs=pl.BlockSpec((B, OUT), lambda i: (0, 0)),
        out_shape=jax.ShapeDtypeStruct((B, OUT), jnp.float32),
        scratch_shapes=[
            pltpu.VMEM((B, D), jnp.float32),
            pltpu.VMEM((1, B), jnp.float32),
        ],
    )(aggp, u, degp, b, g, be, batch.reshape(nblk, 1, R), graph_attr,
      fc1_W, fc1_b, fc2_W, fc2_b)


def kernel(x, edge_index, batch, graph_attr, W1, b1, W2, b2, W3, b3,
           g1, be1, g2, be2, g3, be3, fc1_W, fc1_b, fc2_W, fc2_b):
    N, D = x.shape
    E = edge_index.shape[1]
    src = edge_index[0]
    dst = edge_index[1]

    deg_k = _make_deg_kernel(N, E)
    agg_k = _make_agg_kernel(N, E, D)

    degp = deg_k(dst)
    u1 = _pre_tc(x, W1, degp)
    a1 = agg_k(u1, src, dst)
    u2 = _mid_tc(a1, u1, degp, b1, g1, be1, W2)
    a2 = agg_k(u2, src, dst)
    u3 = _mid_tc(a2, u2, degp, b2, g2, be2, W3)
    a3 = agg_k(u3, src, dst)
    return _final_tc(a3, u3, degp, b3, g3, be3, batch, graph_attr,
                     fc1_W, fc1_b, fc2_W, fc2_b)
